# baseline (speedup 1.0000x reference)
import functools

import jax
import jax.numpy as jnp
from jax import lax
from jax.experimental import pallas as pl
from jax.experimental.pallas import tpu as pltpu
from jax.experimental.pallas import tpu_sc as plsc

N = 10000
NPAD = 10112
DUMP = N
E = 320000
NC = 2
NS = 16
NW = NC * NS
K = 128
K2 = 64
EPT = 10240
CH = EPT // K2
CHH = CH // 2
EPAD = NW * EPT
NSTR = NPAD // NS


def _mp_kernel(D):
    mesh = plsc.VectorSubcoreMesh(core_axis_name="c", subcore_axis_name="s")

    @functools.partial(
        pl.kernel,
        out_type=jax.ShapeDtypeStruct((2 * NPAD, D), jnp.float32),
        mesh=mesh,
        scratch_types=[
            pltpu.VMEM((CHH, K2), jnp.int32),
            pltpu.VMEM((CHH, K2), jnp.int32),
            pltpu.VMEM((K, D), jnp.float32),
            pltpu.VMEM_SHARED((NPAD, D), jnp.float32),
            pltpu.SemaphoreType.DMA,
            pltpu.SemaphoreType.DMA,
            pltpu.SemaphoreType.DMA,
            pltpu.SemaphoreType.DMA,
        ],
    )
    def mp(t_hbm, gidx_hbm, sidx_hbm, zeros_hbm, out_hbm,
           gi_v, si_v, rows_v, acc_sh, *sems):
        gsem = sems[:2]
        ssem = sems[2:]
        cid = lax.axis_index("c")
        sid = lax.axis_index("s")
        wid = cid * NS + sid
        base = sid * NSTR
        pltpu.sync_copy(zeros_hbm, acc_sh.at[pl.ds(base, NSTR)])
        plsc.subcore_barrier()

        qs = [rows_v.at[pl.ds(b * K2, K2)] for b in range(2)]

        def body(i, carry):
            chs = [2 * i + b for b in range(2)]
            gd = [pltpu.async_copy(t_hbm.at[gi_v.at[chs[b]]], qs[b], gsem[b])
                  for b in range(2)]
            sd = []
            for b in range(2):
                gd[b].wait()
                sd.append(pltpu.async_copy(
                    qs[b], acc_sh.at[si_v.at[chs[b]]], ssem[b], add=True))
            for b in range(2):
                sd[b].wait()
            return carry

        for ph in range(2):
            pltpu.sync_copy(gidx_hbm.at[wid, pl.ds(ph * CHH, CHH)], gi_v)
            pltpu.sync_copy(sidx_hbm.at[wid, pl.ds(ph * CHH, CHH)], si_v)
            lax.fori_loop(0, CHH // 2, body, 0, unroll=False)
        plsc.subcore_barrier()
        pltpu.sync_copy(acc_sh.at[pl.ds(base, NSTR)],
                        out_hbm.at[pl.ds(cid * NPAD + base, NSTR)])

    return mp


def _sig(att_ref, i):
    return jax.nn.sigmoid(att_ref[0:1, i:i + 1])


def _stage_a(dT0, dT1, x, w0, r_o, t0_o):
    dr = dT0[:, 0:1] + dT1[:, 0:1]
    r = lax.rsqrt(jnp.maximum(dr, 1.0))
    r_o[...] = r
    t = jnp.dot(x[...] * r, w0[...], preferred_element_type=jnp.float32)
    t0_o[...] = jnp.concatenate(
        [t[:, :64], jnp.ones((NPAD, 1), jnp.float32),
         jnp.zeros((NPAD, 63), jnp.float32)], axis=1)


def _stage_even_epi0(a0, a1, b, att, q_o, m_o):
    s = _sig(att, 0)
    agg = a0[...] + a1[...]
    q = lax.rsqrt(jnp.maximum(agg[:, 64:65], 1.0))
    q_o[...] = q
    h = jnp.maximum(s * (q * agg + b[...]), 0.0)
    m = q * h
    m_o[...] = jnp.concatenate(
        [m[:, :64], jnp.zeros((NPAD, 64), jnp.float32)], axis=1)


def _stage_even_epi(i, a0, a1, q, b, att, m_o):
    s = _sig(att, i)
    q_ = q[...]
    h = jnp.maximum(s * (q_ * (a0[...] + a1[...]) + b[...]), 0.0)
    m_o[...] = q_ * h


def _stage_odd_epi(i, u0, u1, r, w, b, wn, att, t_o):
    s = _sig(att, i)
    r_ = r[...]
    h = jnp.maximum(
        s * (jnp.dot(r_ * (u0[...] + u1[...]), w[...],
                     preferred_element_type=jnp.float32) + b[...]), 0.0)
    t_o[...] = jnp.dot(h * r_, wn[...], preferred_element_type=jnp.float32)


def _stage_final(u0, u1, r, w5, b5, fcw, fcb, att, out_o):
    s = _sig(att, 5)
    r_ = r[...]
    h = jnp.maximum(
        s * (jnp.dot(r_ * (u0[...] + u1[...]), w5[...],
                     preferred_element_type=jnp.float32) + b5[...]), 0.0)
    out_o[...] = jnp.dot(h, fcw[...], preferred_element_type=jnp.float32) + fcb[...]


def _tc(body, out_shapes, *args):
    return pl.pallas_call(body, out_shape=out_shapes)(*args)


def kernel(x, edge_index, W0, b0, W1, b1, W2, b2, W3, b3, W4, b4, W5, b5,
           att, fcW, fcb):
    f32 = jnp.float32
    src = edge_index[0].astype(jnp.int32)
    dst = edge_index[1].astype(jnp.int32)
    npad_edges = EPAD - E
    zpad = jnp.zeros((npad_edges,), jnp.int32)
    dpad = jnp.full((npad_edges,), DUMP, jnp.int32)
    oA = jnp.argsort(src)
    oT = jnp.argsort(dst)
    gA = jnp.concatenate([src[oA], zpad]).reshape(NW, CH, K2)
    sA = jnp.concatenate([dst[oA], dpad]).reshape(NW, CH, K2)
    gT = jnp.concatenate([dst[oT], zpad]).reshape(NW, CH, K2)
    sT = jnp.concatenate([src[oT], dpad]).reshape(NW, CH, K2)

    zeros128 = jnp.zeros((NSTR, 128), f32)
    W0p = jnp.pad(W0, ((0, 0), (0, 64)))
    W1p = jnp.pad(W1, ((0, 64), (0, 0)))
    b0p = jnp.pad(b0, (0, 64))
    ones_t = jnp.ones((NPAD, 128), f32)
    x_p = jnp.pad(x, ((0, NPAD - N), (0, 0)))
    att2 = jnp.pad(att.astype(f32), (0, 2)).reshape(1, 8)
    b1_, b2_, b3_ = b1.reshape(1, -1), b2.reshape(1, -1), b3.reshape(1, -1)
    b4_, b5_ = b4.reshape(1, -1), b5.reshape(1, -1)
    fcb_ = fcb.reshape(1, -1)

    mp128 = _mp_kernel(128)

    dT = mp128(ones_t, gT, sT, zeros128)

    sds = jax.ShapeDtypeStruct
    r, t0 = _tc(_stage_a,
                (sds((NPAD, 1), f32), sds((NPAD, 128), f32)),
                dT[:NPAD], dT[NPAD:], x_p, W0p)

    p = mp128(t0, gA, sA, zeros128)
    q, m1 = _tc(_stage_even_epi0,
                (sds((NPAD, 1), f32), sds((NPAD, 128), f32)),
                p[:NPAD], p[NPAD:], b0p.reshape(1, -1), att2)
    p = mp128(m1, gT, sT, zeros128)
    t2 = _tc(functools.partial(_stage_odd_epi, 1),
             sds((NPAD, 128), f32), p[:NPAD], p[NPAD:], r, W1p, b1_, W2, att2)
    p = mp128(t2, gA, sA, zeros128)
    m3 = _tc(functools.partial(_stage_even_epi, 2),
             sds((NPAD, 128), f32), p[:NPAD], p[NPAD:], q, b2_, att2)
    p = mp128(m3, gT, sT, zeros128)
    t4 = _tc(functools.partial(_stage_odd_epi, 3),
             sds((NPAD, 128), f32), p[:NPAD], p[NPAD:], r, W3, b3_, W4, att2)
    p = mp128(t4, gA, sA, zeros128)
    m5 = _tc(functools.partial(_stage_even_epi, 4),
             sds((NPAD, 128), f32), p[:NPAD], p[NPAD:], q, b4_, att2)
    p = mp128(m5, gT, sT, zeros128)
    out = _tc(_stage_final,
              sds((NPAD, 2), f32), p[:NPAD], p[NPAD:], r, W5, b5_, fcW, fcb_,
              att2)
    return out[:N]


# speedup vs baseline: 3.3021x; 3.3021x over previous
import functools

import jax
import jax.numpy as jnp
from jax import lax
from jax.experimental import pallas as pl
from jax.experimental.pallas import tpu as pltpu
from jax.experimental.pallas import tpu_sc as plsc

N = 10000
NPAD = 10112
DUMP = N
E = 320000
NC = 2
NS = 16
NW = NC * NS
K = 128
K2 = 64
EPT = 10240
CH = EPT // K2
CHH = CH // 2
EPAD = NW * EPT
NSTR = NPAD // NS


def _mp_kernel(D):
    mesh = plsc.VectorSubcoreMesh(core_axis_name="c", subcore_axis_name="s")

    @functools.partial(
        pl.kernel,
        out_type=jax.ShapeDtypeStruct((2 * NPAD, D), jnp.float32),
        mesh=mesh,
        scratch_types=[
            pltpu.VMEM((CHH, K2), jnp.int32),
            pltpu.VMEM((CHH, K2), jnp.int32),
            pltpu.VMEM((K, D), jnp.float32),
            pltpu.VMEM_SHARED((NPAD, D), jnp.float32),
            pltpu.SemaphoreType.DMA,
            pltpu.SemaphoreType.DMA,
            pltpu.SemaphoreType.DMA,
            pltpu.SemaphoreType.DMA,
        ],
    )
    def mp(t_hbm, gidx_hbm, sidx_hbm, zeros_hbm, out_hbm,
           gi_v, si_v, rows_v, acc_sh, *sems):
        gsem = sems[:2]
        ssem = sems[2:]
        cid = lax.axis_index("c")
        sid = lax.axis_index("s")
        wid = cid * NS + sid
        base = sid * NSTR
        pltpu.sync_copy(zeros_hbm, acc_sh.at[pl.ds(base, NSTR)])
        plsc.subcore_barrier()

        qs = [rows_v.at[pl.ds(b * K2, K2)] for b in range(2)]

        def body(i, carry):
            chs = [2 * i + b for b in range(2)]
            gd = [pltpu.async_copy(t_hbm.at[gi_v.at[chs[b]]], qs[b], gsem[b])
                  for b in range(2)]
            sd = []
            for b in range(2):
                gd[b].wait()
                sd.append(pltpu.async_copy(
                    qs[b], acc_sh.at[si_v.at[chs[b]]], ssem[b], add=True))
            for b in range(2):
                sd[b].wait()
            return carry

        for ph in range(2):
            pltpu.sync_copy(gidx_hbm.at[wid, pl.ds(ph * CHH, CHH)], gi_v)
            pltpu.sync_copy(sidx_hbm.at[wid, pl.ds(ph * CHH, CHH)], si_v)
            lax.fori_loop(0, CHH // 2, body, 0, unroll=False)
        plsc.subcore_barrier()
        pltpu.sync_copy(acc_sh.at[pl.ds(base, NSTR)],
                        out_hbm.at[pl.ds(cid * NPAD + base, NSTR)])

    return mp


def _sig(att_ref, i):
    return jax.nn.sigmoid(att_ref[0:1, i:i + 1])


def _stage_a(dT0, dT1, x, w0, r_o, t0_o):
    dr = dT0[:, 0:1] + dT1[:, 0:1]
    r = lax.rsqrt(jnp.maximum(dr, 1.0))
    r_o[...] = r
    t = jnp.dot(x[...] * r, w0[...], preferred_element_type=jnp.float32)
    t0_o[...] = jnp.concatenate(
        [t[:, :64], jnp.ones((NPAD, 1), jnp.float32),
         jnp.zeros((NPAD, 63), jnp.float32)], axis=1)


def _stage_even_epi0(a0, a1, b, att, q_o, m_o):
    s = _sig(att, 0)
    agg = a0[...] + a1[...]
    q = lax.rsqrt(jnp.maximum(agg[:, 64:65], 1.0))
    q_o[...] = q
    h = jnp.maximum(s * (q * agg + b[...]), 0.0)
    m = q * h
    m_o[...] = jnp.concatenate(
        [m[:, :64], jnp.zeros((NPAD, 64), jnp.float32)], axis=1)


def _stage_even_epi(i, a0, a1, q, b, att, m_o):
    s = _sig(att, i)
    q_ = q[...]
    h = jnp.maximum(s * (q_ * (a0[...] + a1[...]) + b[...]), 0.0)
    m_o[...] = q_ * h


def _stage_odd_epi(i, u0, u1, r, w, b, wn, att, t_o):
    s = _sig(att, i)
    r_ = r[...]
    h = jnp.maximum(
        s * (jnp.dot(r_ * (u0[...] + u1[...]), w[...],
                     preferred_element_type=jnp.float32) + b[...]), 0.0)
    t_o[...] = jnp.dot(h * r_, wn[...], preferred_element_type=jnp.float32)


def _stage_final(u0, u1, r, w5, b5, fcw, fcb, att, out_o):
    s = _sig(att, 5)
    r_ = r[...]
    h = jnp.maximum(
        s * (jnp.dot(r_ * (u0[...] + u1[...]), w5[...],
                     preferred_element_type=jnp.float32) + b5[...]), 0.0)
    out_o[...] = jnp.dot(h, fcw[...], preferred_element_type=jnp.float32) + fcb[...]


def _tc(body, out_shapes, *args):
    return pl.pallas_call(body, out_shape=out_shapes)(*args)


def kernel(x, edge_index, W0, b0, W1, b1, W2, b2, W3, b3, W4, b4, W5, b5,
           att, fcW, fcb):
    f32 = jnp.float32
    src = edge_index[0].astype(jnp.int32)
    dst = edge_index[1].astype(jnp.int32)
    npad_edges = EPAD - E
    par = jnp.arange(npad_edges, dtype=jnp.int32)
    zpad = par % N
    dpad = N + par % (NPAD - N)
    gA = jnp.concatenate([src, zpad]).reshape(NW, CH, K2)
    sA = jnp.concatenate([dst, dpad]).reshape(NW, CH, K2)
    gT = jnp.concatenate([dst, zpad]).reshape(NW, CH, K2)
    sT = jnp.concatenate([src, dpad]).reshape(NW, CH, K2)

    zeros128 = jnp.zeros((NSTR, 128), f32)
    W0p = jnp.pad(W0, ((0, 0), (0, 64)))
    W1p = jnp.pad(W1, ((0, 64), (0, 0)))
    b0p = jnp.pad(b0, (0, 64))
    ones_t = jnp.ones((NPAD, 128), f32)
    x_p = jnp.pad(x, ((0, NPAD - N), (0, 0)))
    att2 = jnp.pad(att.astype(f32), (0, 2)).reshape(1, 8)
    b1_, b2_, b3_ = b1.reshape(1, -1), b2.reshape(1, -1), b3.reshape(1, -1)
    b4_, b5_ = b4.reshape(1, -1), b5.reshape(1, -1)
    fcb_ = fcb.reshape(1, -1)

    mp128 = _mp_kernel(128)

    dT = mp128(ones_t, gT, sT, zeros128)

    sds = jax.ShapeDtypeStruct
    r, t0 = _tc(_stage_a,
                (sds((NPAD, 1), f32), sds((NPAD, 128), f32)),
                dT[:NPAD], dT[NPAD:], x_p, W0p)

    p = mp128(t0, gA, sA, zeros128)
    q, m1 = _tc(_stage_even_epi0,
                (sds((NPAD, 1), f32), sds((NPAD, 128), f32)),
                p[:NPAD], p[NPAD:], b0p.reshape(1, -1), att2)
    p = mp128(m1, gT, sT, zeros128)
    t2 = _tc(functools.partial(_stage_odd_epi, 1),
             sds((NPAD, 128), f32), p[:NPAD], p[NPAD:], r, W1p, b1_, W2, att2)
    p = mp128(t2, gA, sA, zeros128)
    m3 = _tc(functools.partial(_stage_even_epi, 2),
             sds((NPAD, 128), f32), p[:NPAD], p[NPAD:], q, b2_, att2)
    p = mp128(m3, gT, sT, zeros128)
    t4 = _tc(functools.partial(_stage_odd_epi, 3),
             sds((NPAD, 128), f32), p[:NPAD], p[NPAD:], r, W3, b3_, W4, att2)
    p = mp128(t4, gA, sA, zeros128)
    m5 = _tc(functools.partial(_stage_even_epi, 4),
             sds((NPAD, 128), f32), p[:NPAD], p[NPAD:], q, b4_, att2)
    p = mp128(m5, gT, sT, zeros128)
    out = _tc(_stage_final,
              sds((NPAD, 2), f32), p[:NPAD], p[NPAD:], r, W5, b5_, fcW, fcb_,
              att2)
    return out[:N]


# speedup vs baseline: 3.5740x; 1.0823x over previous
import functools

import jax
import jax.numpy as jnp
from jax import lax
from jax.experimental import pallas as pl
from jax.experimental.pallas import tpu as pltpu
from jax.experimental.pallas import tpu_sc as plsc

N = 10000
NPAD = 10112
DUMP = N
E = 320000
NC = 2
NS = 16
NW = NC * NS
K = 128
K2 = 64
EPT = 10240
CH = EPT // K2
CHH = CH // 2
EPAD = NW * EPT
NSTR = NPAD // NS


def _mp_kernel(D):
    mesh = plsc.VectorSubcoreMesh(core_axis_name="c", subcore_axis_name="s")

    @functools.partial(
        pl.kernel,
        out_type=jax.ShapeDtypeStruct((2 * NPAD, D), jnp.float32),
        mesh=mesh,
        scratch_types=[
            pltpu.VMEM((CHH, K2), jnp.int32),
            pltpu.VMEM((CHH, K2), jnp.int32),
            pltpu.VMEM((K, D), jnp.float32),
            pltpu.VMEM_SHARED((NPAD, D), jnp.float32),
            pltpu.SemaphoreType.DMA,
            pltpu.SemaphoreType.DMA,
            pltpu.SemaphoreType.DMA,
            pltpu.SemaphoreType.DMA,
        ],
    )
    def mp(t_hbm, gidx_hbm, sidx_hbm, zeros_hbm, out_hbm,
           gi_v, si_v, rows_v, acc_sh, *sems):
        gsem = sems[:2]
        ssem = sems[2:]
        cid = lax.axis_index("c")
        sid = lax.axis_index("s")
        wid = cid * NS + sid
        base = sid * NSTR
        pltpu.sync_copy(zeros_hbm, acc_sh.at[pl.ds(base, NSTR)])
        plsc.subcore_barrier()

        qs = [rows_v.at[pl.ds(b * K2, K2)] for b in range(2)]

        def body(i, carry):
            chs = [2 * i + b for b in range(2)]
            gd = [pltpu.async_copy(t_hbm.at[gi_v.at[chs[b]]], qs[b], gsem[b])
                  for b in range(2)]
            sd = []
            for b in range(2):
                gd[b].wait()
                sd.append(pltpu.async_copy(
                    qs[b], acc_sh.at[si_v.at[chs[b]]], ssem[b], add=True))
            for b in range(2):
                sd[b].wait()
            return carry

        for ph in range(2):
            pltpu.sync_copy(gidx_hbm.at[wid, pl.ds(ph * CHH, CHH)], gi_v)
            pltpu.sync_copy(sidx_hbm.at[wid, pl.ds(ph * CHH, CHH)], si_v)
            lax.fori_loop(0, CHH // 2, body, 0, unroll=False)
        plsc.subcore_barrier()
        pltpu.sync_copy(acc_sh.at[pl.ds(base, NSTR)],
                        out_hbm.at[pl.ds(cid * NPAD + base, NSTR)])

    return mp


def _deg_kernel():
    mesh = plsc.VectorSubcoreMesh(core_axis_name="c", subcore_axis_name="s")

    @functools.partial(
        pl.kernel,
        out_type=jax.ShapeDtypeStruct((2 * NPAD, 128), jnp.float32),
        mesh=mesh,
        scratch_types=[
            pltpu.VMEM((CHH, K2), jnp.int32),
            pltpu.VMEM((K2, 128), jnp.float32),
            pltpu.VMEM_SHARED((NPAD, 128), jnp.float32),
        ],
    )
    def deg(sidx_hbm, ones_hbm, zeros_hbm, out_hbm, si_v, ones_v, acc_sh):
        cid = lax.axis_index("c")
        sid = lax.axis_index("s")
        wid = cid * NS + sid
        base = sid * NSTR
        pltpu.sync_copy(zeros_hbm, acc_sh.at[pl.ds(base, NSTR)])
        pltpu.sync_copy(ones_hbm, ones_v)
        plsc.subcore_barrier()

        def body(ch, carry):
            pltpu.sync_copy(ones_v, acc_sh.at[si_v.at[ch]], add=True)
            return carry

        for ph in range(2):
            pltpu.sync_copy(sidx_hbm.at[wid, pl.ds(ph * CHH, CHH)], si_v)
            lax.fori_loop(0, CHH, body, 0, unroll=False)
        plsc.subcore_barrier()
        pltpu.sync_copy(acc_sh.at[pl.ds(base, NSTR)],
                        out_hbm.at[pl.ds(cid * NPAD + base, NSTR)])

    return deg


def _sig(att_ref, i):
    return jax.nn.sigmoid(att_ref[0:1, i:i + 1])


def _stage_a(dT0, dT1, x, w0, r_o, t0_o):
    dr = dT0[:, 0:1] + dT1[:, 0:1]
    r = lax.rsqrt(jnp.maximum(dr, 1.0))
    r_o[...] = r
    t = jnp.dot(x[...] * r, w0[...], preferred_element_type=jnp.float32)
    t0_o[...] = jnp.concatenate(
        [t[:, :64], jnp.ones((NPAD, 1), jnp.float32),
         jnp.zeros((NPAD, 63), jnp.float32)], axis=1)


def _stage_even_epi0(a0, a1, b, att, q_o, m_o):
    s = _sig(att, 0)
    agg = a0[...] + a1[...]
    q = lax.rsqrt(jnp.maximum(agg[:, 64:65], 1.0))
    q_o[...] = q
    h = jnp.maximum(s * (q * agg + b[...]), 0.0)
    m = q * h
    m_o[...] = jnp.concatenate(
        [m[:, :64], jnp.zeros((NPAD, 64), jnp.float32)], axis=1)


def _stage_even_epi(i, a0, a1, q, b, att, m_o):
    s = _sig(att, i)
    q_ = q[...]
    h = jnp.maximum(s * (q_ * (a0[...] + a1[...]) + b[...]), 0.0)
    m_o[...] = q_ * h


def _stage_odd_epi(i, u0, u1, r, w, b, wn, att, t_o):
    s = _sig(att, i)
    r_ = r[...]
    h = jnp.maximum(
        s * (jnp.dot(r_ * (u0[...] + u1[...]), w[...],
                     preferred_element_type=jnp.float32) + b[...]), 0.0)
    t_o[...] = jnp.dot(h * r_, wn[...], preferred_element_type=jnp.float32)


def _stage_final(u0, u1, r, w5, b5, fcw, fcb, att, out_o):
    s = _sig(att, 5)
    r_ = r[...]
    h = jnp.maximum(
        s * (jnp.dot(r_ * (u0[...] + u1[...]), w5[...],
                     preferred_element_type=jnp.float32) + b5[...]), 0.0)
    out_o[...] = jnp.dot(h, fcw[...], preferred_element_type=jnp.float32) + fcb[...]


def _tc(body, out_shapes, *args):
    return pl.pallas_call(body, out_shape=out_shapes)(*args)


def kernel(x, edge_index, W0, b0, W1, b1, W2, b2, W3, b3, W4, b4, W5, b5,
           att, fcW, fcb):
    f32 = jnp.float32
    src = edge_index[0].astype(jnp.int32)
    dst = edge_index[1].astype(jnp.int32)
    npad_edges = EPAD - E
    par = jnp.arange(npad_edges, dtype=jnp.int32)
    zpad = par % N
    dpad = N + par % (NPAD - N)
    gA = jnp.concatenate([src, zpad]).reshape(NW, CH, K2)
    sA = jnp.concatenate([dst, dpad]).reshape(NW, CH, K2)
    gT = jnp.concatenate([dst, zpad]).reshape(NW, CH, K2)
    sT = jnp.concatenate([src, dpad]).reshape(NW, CH, K2)

    zeros128 = jnp.zeros((NSTR, 128), f32)
    W0p = jnp.pad(W0, ((0, 0), (0, 64)))
    W1p = jnp.pad(W1, ((0, 64), (0, 0)))
    b0p = jnp.pad(b0, (0, 64))
    ones_k = jnp.ones((K2, 128), f32)
    x_p = jnp.pad(x, ((0, NPAD - N), (0, 0)))
    att2 = jnp.pad(att.astype(f32), (0, 2)).reshape(1, 8)
    b1_, b2_, b3_ = b1.reshape(1, -1), b2.reshape(1, -1), b3.reshape(1, -1)
    b4_, b5_ = b4.reshape(1, -1), b5.reshape(1, -1)
    fcb_ = fcb.reshape(1, -1)

    mp128 = _mp_kernel(128)

    dT = _deg_kernel()(sT, ones_k, zeros128)

    sds = jax.ShapeDtypeStruct
    r, t0 = _tc(_stage_a,
                (sds((NPAD, 1), f32), sds((NPAD, 128), f32)),
                dT[:NPAD], dT[NPAD:], x_p, W0p)

    p = mp128(t0, gA, sA, zeros128)
    q, m1 = _tc(_stage_even_epi0,
                (sds((NPAD, 1), f32), sds((NPAD, 128), f32)),
                p[:NPAD], p[NPAD:], b0p.reshape(1, -1), att2)
    p = mp128(m1, gT, sT, zeros128)
    t2 = _tc(functools.partial(_stage_odd_epi, 1),
             sds((NPAD, 128), f32), p[:NPAD], p[NPAD:], r, W1p, b1_, W2, att2)
    p = mp128(t2, gA, sA, zeros128)
    m3 = _tc(functools.partial(_stage_even_epi, 2),
             sds((NPAD, 128), f32), p[:NPAD], p[NPAD:], q, b2_, att2)
    p = mp128(m3, gT, sT, zeros128)
    t4 = _tc(functools.partial(_stage_odd_epi, 3),
             sds((NPAD, 128), f32), p[:NPAD], p[NPAD:], r, W3, b3_, W4, att2)
    p = mp128(t4, gA, sA, zeros128)
    m5 = _tc(functools.partial(_stage_even_epi, 4),
             sds((NPAD, 128), f32), p[:NPAD], p[NPAD:], q, b4_, att2)
    p = mp128(m5, gT, sT, zeros128)
    out = _tc(_stage_final,
              sds((NPAD, 2), f32), p[:NPAD], p[NPAD:], r, W5, b5_, fcW, fcb_,
              att2)
    return out[:N]
